# Initial kernel scaffold; baseline (speedup 1.0000x reference)
import jax
import jax.numpy as jnp
from jax.experimental import pallas as pl


def kernel(batch_inputs, node_embs, edge_embs, edge_list, node_list, Wn0, We0, ae0, an0, Wn1, We1, ae1, an1):
    raise NotImplementedError("write your pallas kernel here")


# speedup vs baseline: 3.4387x; 3.4387x over previous
import functools

import jax
import jax.numpy as jnp
from jax import lax
from jax.experimental import pallas as pl
from jax.experimental.pallas import tpu as pltpu
from jax.experimental.pallas import tpu_sc as plsc

F = 128
A = 4
L = 16
NC, NS = 2, 16
NW = NC * NS
C = 16
ALPHA = 0.2
BR = 1024


def _bcast16(v, lane):
    idx = jnp.full((L, 1), lane, dtype=jnp.int32)
    return lax.gather(
        v, idx,
        lax.GatherDimensionNumbers(offset_dims=(), collapsed_slice_dims=(0,),
                                   start_index_map=(0,)),
        (1,), mode=lax.GatherScatterMode.PROMISE_IN_BOUNDS)


def _elu(v):
    return jnp.where(v > 0, v, jnp.exp(jnp.minimum(v, 0.0)) - 1.0)


def _node_dense_body(x_ref, W_ref, a1_ref, a2_ref, h_ref, p_ref, q_ref):
    h = jnp.dot(x_ref[...], W_ref[...], preferred_element_type=jnp.float32)
    h_ref[...] = h
    p_ref[...] = jnp.dot(h, a1_ref[...], preferred_element_type=jnp.float32)
    q_ref[...] = jnp.dot(h, a2_ref[...], preferred_element_type=jnp.float32)


@functools.lru_cache(maxsize=None)
def _node_dense_call(R, interpret=False):
    grid = (R + BR - 1) // BR
    return pl.pallas_call(
        _node_dense_body,
        grid=(grid,),
        in_specs=[pl.BlockSpec((BR, F), lambda i: (i, 0)),
                  pl.BlockSpec((F, F), lambda i: (0, 0)),
                  pl.BlockSpec((F, 1), lambda i: (0, 0)),
                  pl.BlockSpec((F, 1), lambda i: (0, 0))],
        out_specs=[pl.BlockSpec((BR, F), lambda i: (i, 0)),
                   pl.BlockSpec((BR, 1), lambda i: (i, 0)),
                   pl.BlockSpec((BR, 1), lambda i: (i, 0))],
        out_shape=[jax.ShapeDtypeStruct((R, F), jnp.float32),
                   jax.ShapeDtypeStruct((R, 1), jnp.float32),
                   jax.ShapeDtypeStruct((R, 1), jnp.float32)],
        interpret=interpret,
    )


def _edge0_body(x_ref, W_ref, a_ref, c_ref):
    w = jnp.dot(W_ref[...], a_ref[...], preferred_element_type=jnp.float32)
    c_ref[...] = jnp.dot(x_ref[...], w, preferred_element_type=jnp.float32)


@functools.lru_cache(maxsize=None)
def _edge0_call(R, interpret=False):
    grid = (R + BR - 1) // BR
    return pl.pallas_call(
        _edge0_body,
        grid=(grid,),
        in_specs=[pl.BlockSpec((BR, F), lambda i: (i, 0)),
                  pl.BlockSpec((F, F), lambda i: (0, 0)),
                  pl.BlockSpec((F, 1), lambda i: (0, 0))],
        out_specs=[pl.BlockSpec((BR, 1), lambda i: (i, 0))],
        out_shape=[jax.ShapeDtypeStruct((R, 1), jnp.float32)],
        interpret=interpret,
    )


def _e1_post_body(e_ref, anb_ref, We_ref, aea_ref, r_ref, c_ref):
    e = e_ref[...]
    r_ref[...] = jnp.dot(e, anb_ref[...], preferred_element_type=jnp.float32)
    ye = jnp.where(e > 0, e, jnp.exp(jnp.minimum(e, 0.0)) - 1.0)
    w = jnp.dot(We_ref[...], aea_ref[...], preferred_element_type=jnp.float32)
    c_ref[...] = jnp.dot(ye, w, preferred_element_type=jnp.float32)


@functools.lru_cache(maxsize=None)
def _e1_post_call(R, interpret=False):
    grid = (R + BR - 1) // BR
    return pl.pallas_call(
        _e1_post_body,
        grid=(grid,),
        in_specs=[pl.BlockSpec((BR, F), lambda i: (i, 0)),
                  pl.BlockSpec((F, 1), lambda i: (0, 0)),
                  pl.BlockSpec((F, F), lambda i: (0, 0)),
                  pl.BlockSpec((F, 1), lambda i: (0, 0))],
        out_specs=[pl.BlockSpec((BR, 1), lambda i: (i, 0)),
                   pl.BlockSpec((BR, 1), lambda i: (i, 0))],
        out_shape=[jax.ShapeDtypeStruct((R, 1), jnp.float32),
                   jax.ShapeDtypeStruct((R, 1), jnp.float32)],
        interpret=interpret,
    )


def _e2_post_body(e_ref, anb_ref, r_ref):
    r_ref[...] = jnp.dot(e_ref[...], anb_ref[...],
                         preferred_element_type=jnp.float32)


@functools.lru_cache(maxsize=None)
def _e2_post_call(R, interpret=False):
    grid = (R + BR - 1) // BR
    return pl.pallas_call(
        _e2_post_body,
        grid=(grid,),
        in_specs=[pl.BlockSpec((BR, F), lambda i: (i, 0)),
                  pl.BlockSpec((F, 1), lambda i: (0, 0))],
        out_specs=[pl.BlockSpec((BR, 1), lambda i: (i, 0))],
        out_shape=[jax.ShapeDtypeStruct((R, 1), jnp.float32)],
        interpret=interpret,
    )


@functools.lru_cache(maxsize=None)
def _sc_stage_call(Mp, Ntab, elu, interpret=False):
    Tw = Mp // NW
    G = Tw // C
    CA = C * A
    mesh = plsc.VectorSubcoreMesh(core_axis_name="c", subcore_axis_name="s", num_cores=NC, num_subcores=NS)

    def body(idx_hbm, c_hbm, p_hbm, v_hbm, out_hbm,
             c_all, idxb, pg, rows, outb, sem1, sem2):
        wid = lax.axis_index("s") * NC + lax.axis_index("c")
        base = pl.multiple_of(wid * Tw, 8)
        pltpu.sync_copy(c_hbm.at[pl.ds(base, Tw)], c_all)
        i16 = lax.iota(jnp.int32, L)
        i4 = i16 * A

        def chunk(g, carry):
            off = pl.multiple_of(base + g * C, 8)
            pltpu.sync_copy(idx_hbm.at[pl.ds(off * A, CA)], idxb)
            cp = pltpu.async_copy(p_hbm.at[idxb], pg, sem1)
            cv = pltpu.async_copy(v_hbm.at[idxb], rows, sem2)
            cp.wait()
            cv.wait()
            for h in range(C // L):
                cvec = c_all[pl.ds(g * C + h * L, L)]
                ps = [plsc.load_gather(pg, [i4 + (h * L * A + m)])
                      for m in range(A)]
                ss = [cvec + p for p in ps]
                ss = [jnp.where(s > 0, s, ALPHA * s) for s in ss]
                mx = jnp.maximum(jnp.maximum(ss[0], ss[1]),
                                 jnp.maximum(ss[2], ss[3]))
                es = [jnp.exp(s - mx) for s in ss]
                tot = (es[0] + es[1]) + (es[2] + es[3])
                att = [e / tot for e in es]
                for i in range(L):
                    ii = h * L + i
                    ws = [_bcast16(att[m], i) for m in range(A)]
                    for k in range(F // L):
                        sl = pl.ds(k * L, L)
                        acc = ((rows[ii * A + 0, sl] * ws[0]
                                + rows[ii * A + 1, sl] * ws[1])
                               + (rows[ii * A + 2, sl] * ws[2]
                                  + rows[ii * A + 3, sl] * ws[3]))
                        if elu:
                            acc = _elu(acc)
                        outb[ii, sl] = acc
            pltpu.sync_copy(outb, out_hbm.at[pl.ds(off, C)])
            return carry

        lax.fori_loop(0, G, chunk, 0)

    return pl.kernel(
        body,
        out_type=jax.ShapeDtypeStruct((Mp, F), jnp.float32),
        mesh=mesh,
        scratch_types=[pltpu.VMEM((Tw,), jnp.float32),
                       pltpu.VMEM((CA,), jnp.int32),
                       pltpu.VMEM((CA,), jnp.float32),
                       pltpu.VMEM((CA, F), jnp.float32),
                       pltpu.VMEM((C, F), jnp.float32),
                       pltpu.SemaphoreType.DMA,
                       pltpu.SemaphoreType.DMA],
        compiler_params=pltpu.CompilerParams(needs_layout_passes=False),
        interpret=interpret,
    )


def _sc_stage(idx, cself, ptab, vtab, elu, interpret=False):
    M = idx.shape[0]
    Mp = -(-M // (NW * C)) * (NW * C)
    idx_flat = jnp.pad(idx, ((0, Mp - M), (0, 0))).reshape(-1)
    c_pad = jnp.pad(cself, (0, Mp - cself.shape[0]))
    fn = _sc_stage_call(Mp, vtab.shape[0], elu, interpret)
    return fn(idx_flat, c_pad, ptab, vtab)


@functools.lru_cache(maxsize=None)
def _assembly_call(B, W, Mp, Np, interpret=False):
    Rt = B // NW
    SLT = Rt * W
    mesh = plsc.VectorSubcoreMesh(core_axis_name="c", subcore_axis_name="s", num_cores=NC, num_subcores=NS)

    def body(b_hbm, e_hbm, n_hbm, out_hbm, b_v, idxs, er, nr, outb, sem):
        wid = lax.axis_index("s") * NC + lax.axis_index("c")
        sbase = pl.multiple_of(wid * SLT, 8)
        pltpu.sync_copy(b_hbm.at[pl.ds(sbase, SLT)], b_v)
        i16 = lax.iota(jnp.int32, L)
        i5 = i16 * W
        for h in range(Rt // L):
            hb = h * L * W
            cols = [plsc.load_gather(b_v, [i5 + (hb + j)]) for j in range(W)]
            lastnz = jnp.full((L,), -1, jnp.int32)
            for j in range(W - 1):
                lastnz = jnp.maximum(lastnz,
                                     jnp.where(cols[j] != 0, j, -1))
            right = lastnz + 1
            idxs[...] = cols[0]
            pltpu.async_copy(e_hbm.at[idxs], er, sem).wait()
            for i in range(L):
                s = (h * L + i) * W
                for k in range(F // L):
                    sl = pl.ds(k * L, L)
                    outb[s, sl] = _elu(er[i, sl])
            for j in range(1, W):
                mf = jnp.where(j < right, 1.0, 0.0)
                idxs[...] = cols[j]
                pltpu.async_copy(n_hbm.at[idxs], nr, sem).wait()
                for i in range(L):
                    w = _bcast16(mf, i)
                    s = (h * L + i) * W + j
                    for k in range(F // L):
                        sl = pl.ds(k * L, L)
                        outb[s, sl] = nr[i, sl] * w
        pltpu.sync_copy(outb, out_hbm.at[pl.ds(sbase, SLT)])

    return pl.kernel(
        body,
        out_type=jax.ShapeDtypeStruct((B * W, F), jnp.float32),
        mesh=mesh,
        scratch_types=[pltpu.VMEM((SLT,), jnp.int32),
                       pltpu.VMEM((L,), jnp.int32),
                       pltpu.VMEM((L, F), jnp.float32),
                       pltpu.VMEM((L, F), jnp.float32),
                       pltpu.VMEM((SLT, F), jnp.float32),
                       pltpu.SemaphoreType.DMA],
        compiler_params=pltpu.CompilerParams(needs_layout_passes=False),
        interpret=interpret,
    )


def kernel(batch_inputs, node_embs, edge_embs, edge_list, node_list,
           Wn0, We0, ae0, an0, Wn1, We1, ae1, an1, _interpret=False):
    N = node_embs.shape[0]
    M = edge_embs.shape[0]
    B, W = batch_inputs.shape

    col = lambda a: a.reshape(F, 1)
    hn0, pe0, q0 = _node_dense_call(N, _interpret)(node_embs, Wn0, col(ae0[F:]),
                                       col(an0[:F]))
    (ce0,) = _edge0_call(M, _interpret)(edge_embs, We0, col(ae0[:F]))
    e1 = _sc_stage(edge_list, ce0[:, 0], pe0[:, 0], hn0, False, _interpret)
    r0, ce1 = _e1_post_call(e1.shape[0], _interpret)(e1, col(an0[F:]), We1, col(ae1[:F]))
    x = _sc_stage(node_list, q0[:, 0], r0[:, 0], e1, True, _interpret)
    hn1, pe1, q1 = _node_dense_call(x.shape[0], _interpret)(x, Wn1, col(ae1[F:]),
                                                col(an1[:F]))
    e2 = _sc_stage(edge_list, ce1[:, 0], pe1[:, 0], hn1, False, _interpret)
    (r1,) = _e2_post_call(e2.shape[0], _interpret)(e2, col(an1[F:]))
    nf = _sc_stage(node_list, q1[:, 0], r1[:, 0], e2, True, _interpret)
    out = _assembly_call(B, W, e2.shape[0], nf.shape[0], _interpret)(
        batch_inputs.reshape(-1), e2, nf)
    return out.reshape(B, W, F)


# speedup vs baseline: 4.6280x; 1.3458x over previous
import functools

import jax
import jax.numpy as jnp
from jax import lax
from jax.experimental import pallas as pl
from jax.experimental.pallas import tpu as pltpu
from jax.experimental.pallas import tpu_sc as plsc

F = 128
A = 4
L = 16
NC, NS = 2, 16
NW = NC * NS
C = 16
ALPHA = 0.2
BR = 1024


def _bcast16(v, lane):
    idx = jnp.full((L, 1), lane, dtype=jnp.int32)
    return lax.gather(
        v, idx,
        lax.GatherDimensionNumbers(offset_dims=(), collapsed_slice_dims=(0,),
                                   start_index_map=(0,)),
        (1,), mode=lax.GatherScatterMode.PROMISE_IN_BOUNDS)


def _elu(v):
    return jnp.where(v > 0, v, jnp.exp(jnp.minimum(v, 0.0)) - 1.0)


def _node_dense_body(x_ref, W_ref, a1_ref, a2_ref, h_ref, p_ref, q_ref):
    h = jnp.dot(x_ref[...], W_ref[...], preferred_element_type=jnp.float32)
    h_ref[...] = h
    p_ref[...] = jnp.dot(h, a1_ref[...], preferred_element_type=jnp.float32)
    q_ref[...] = jnp.dot(h, a2_ref[...], preferred_element_type=jnp.float32)


@functools.lru_cache(maxsize=None)
def _node_dense_call(R, interpret=False):
    grid = (R + BR - 1) // BR
    return pl.pallas_call(
        _node_dense_body,
        grid=(grid,),
        in_specs=[pl.BlockSpec((BR, F), lambda i: (i, 0)),
                  pl.BlockSpec((F, F), lambda i: (0, 0)),
                  pl.BlockSpec((F, 1), lambda i: (0, 0)),
                  pl.BlockSpec((F, 1), lambda i: (0, 0))],
        out_specs=[pl.BlockSpec((BR, F), lambda i: (i, 0)),
                   pl.BlockSpec((BR, 1), lambda i: (i, 0)),
                   pl.BlockSpec((BR, 1), lambda i: (i, 0))],
        out_shape=[jax.ShapeDtypeStruct((R, F), jnp.float32),
                   jax.ShapeDtypeStruct((R, 1), jnp.float32),
                   jax.ShapeDtypeStruct((R, 1), jnp.float32)],
        interpret=interpret,
    )


def _edge0_body(x_ref, W_ref, a_ref, c_ref):
    w = jnp.dot(W_ref[...], a_ref[...], preferred_element_type=jnp.float32)
    c_ref[...] = jnp.dot(x_ref[...], w, preferred_element_type=jnp.float32)


@functools.lru_cache(maxsize=None)
def _edge0_call(R, interpret=False):
    grid = (R + BR - 1) // BR
    return pl.pallas_call(
        _edge0_body,
        grid=(grid,),
        in_specs=[pl.BlockSpec((BR, F), lambda i: (i, 0)),
                  pl.BlockSpec((F, F), lambda i: (0, 0)),
                  pl.BlockSpec((F, 1), lambda i: (0, 0))],
        out_specs=[pl.BlockSpec((BR, 1), lambda i: (i, 0))],
        out_shape=[jax.ShapeDtypeStruct((R, 1), jnp.float32)],
        interpret=interpret,
    )


def _e1_post_body(e_ref, anb_ref, We_ref, aea_ref, r_ref, c_ref):
    e = e_ref[...]
    r_ref[...] = jnp.dot(e, anb_ref[...], preferred_element_type=jnp.float32)
    ye = jnp.where(e > 0, e, jnp.exp(jnp.minimum(e, 0.0)) - 1.0)
    w = jnp.dot(We_ref[...], aea_ref[...], preferred_element_type=jnp.float32)
    c_ref[...] = jnp.dot(ye, w, preferred_element_type=jnp.float32)


@functools.lru_cache(maxsize=None)
def _e1_post_call(R, interpret=False):
    grid = (R + BR - 1) // BR
    return pl.pallas_call(
        _e1_post_body,
        grid=(grid,),
        in_specs=[pl.BlockSpec((BR, F), lambda i: (i, 0)),
                  pl.BlockSpec((F, 1), lambda i: (0, 0)),
                  pl.BlockSpec((F, F), lambda i: (0, 0)),
                  pl.BlockSpec((F, 1), lambda i: (0, 0))],
        out_specs=[pl.BlockSpec((BR, 1), lambda i: (i, 0)),
                   pl.BlockSpec((BR, 1), lambda i: (i, 0))],
        out_shape=[jax.ShapeDtypeStruct((R, 1), jnp.float32),
                   jax.ShapeDtypeStruct((R, 1), jnp.float32)],
        interpret=interpret,
    )


def _e2_post_body(e_ref, anb_ref, r_ref):
    r_ref[...] = jnp.dot(e_ref[...], anb_ref[...],
                         preferred_element_type=jnp.float32)


@functools.lru_cache(maxsize=None)
def _e2_post_call(R, interpret=False):
    grid = (R + BR - 1) // BR
    return pl.pallas_call(
        _e2_post_body,
        grid=(grid,),
        in_specs=[pl.BlockSpec((BR, F), lambda i: (i, 0)),
                  pl.BlockSpec((F, 1), lambda i: (0, 0))],
        out_specs=[pl.BlockSpec((BR, 1), lambda i: (i, 0))],
        out_shape=[jax.ShapeDtypeStruct((R, 1), jnp.float32)],
        interpret=interpret,
    )


@functools.lru_cache(maxsize=None)
def _sc_stage_call(Mp, Ntab, elu, interpret=False):
    Tw = Mp // NW
    G = Tw // C
    K = 4
    NG = G // K
    CA = C * A
    mesh = plsc.VectorSubcoreMesh(core_axis_name="c", subcore_axis_name="s",
                                  num_cores=NC, num_subcores=NS)

    def body(idx_hbm, c_hbm, p_hbm, v_hbm, out_hbm,
             c_all, idx_all, pg0, pg1, rows0, rows1, outb,
             semp0, semp1, semv0, semv1):
        pg = (pg0, pg1)
        rows = (rows0, rows1)
        semp = (semp0, semp1)
        semv = (semv0, semv1)
        wid = lax.axis_index("s") * NC + lax.axis_index("c")
        base = pl.multiple_of(wid * Tw, 8)
        pltpu.sync_copy(c_hbm.at[pl.ds(base, Tw)], c_all)
        pltpu.sync_copy(idx_hbm.at[pl.ds(base * A, Tw * A)], idx_all)
        i16 = lax.iota(jnp.int32, L)
        i4 = i16 * A

        def isl(g):
            return idx_all.at[pl.ds(pl.multiple_of(g * CA, 8), CA)]

        def issue(g, b):
            pltpu.async_copy(p_hbm.at[isl(g)], pg[b], semp[b])
            pltpu.async_copy(v_hbm.at[isl(g)], rows[b], semv[b])

        issue(0, 0)

        def group(g2, carry):
            for k in range(K):
                g = g2 * K + k
                b = k & 1
                nb = (k + 1) & 1
                if k < K - 1:
                    issue(g + 1, nb)
                else:
                    @pl.when(g2 < NG - 1)
                    def _():
                        issue((g2 + 1) * K, 0)
                pltpu.make_async_copy(p_hbm.at[isl(g)], pg[b], semp[b]).wait()
                pltpu.make_async_copy(v_hbm.at[isl(g)], rows[b],
                                      semv[b]).wait()
                for h in range(C // L):
                    cvec = c_all[pl.ds(g * C + h * L, L)]
                    ps = [plsc.load_gather(pg[b], [i4 + (h * L * A + m)])
                          for m in range(A)]
                    ss = [cvec + p for p in ps]
                    ss = [jnp.where(s > 0, s, ALPHA * s) for s in ss]
                    mx = jnp.maximum(jnp.maximum(ss[0], ss[1]),
                                     jnp.maximum(ss[2], ss[3]))
                    es = [jnp.exp(s - mx) for s in ss]
                    tot = (es[0] + es[1]) + (es[2] + es[3])
                    att = [e / tot for e in es]
                    for i in range(L):
                        ii = h * L + i
                        ws = [_bcast16(att[m], i) for m in range(A)]
                        for kk in range(F // L):
                            sl = pl.ds(kk * L, L)
                            acc = ((rows[b][ii * A + 0, sl] * ws[0]
                                    + rows[b][ii * A + 1, sl] * ws[1])
                                   + (rows[b][ii * A + 2, sl] * ws[2]
                                      + rows[b][ii * A + 3, sl] * ws[3]))
                            if elu:
                                acc = _elu(acc)
                            outb[k * C + ii, sl] = acc
            pltpu.sync_copy(outb,
                            out_hbm.at[pl.ds(base + g2 * (K * C), K * C)])
            return carry

        lax.fori_loop(0, NG, group, 0)

    return pl.kernel(
        body,
        out_type=jax.ShapeDtypeStruct((Mp, F), jnp.float32),
        mesh=mesh,
        scratch_types=[pltpu.VMEM((Tw,), jnp.float32),
                       pltpu.VMEM((Tw * A,), jnp.int32),
                       pltpu.VMEM((CA,), jnp.float32),
                       pltpu.VMEM((CA,), jnp.float32),
                       pltpu.VMEM((CA, F), jnp.float32),
                       pltpu.VMEM((CA, F), jnp.float32),
                       pltpu.VMEM((K * C, F), jnp.float32),
                       pltpu.SemaphoreType.DMA,
                       pltpu.SemaphoreType.DMA,
                       pltpu.SemaphoreType.DMA,
                       pltpu.SemaphoreType.DMA],
        compiler_params=pltpu.CompilerParams(needs_layout_passes=False),
        interpret=interpret,
    )


def _sc_stage(idx, cself, ptab, vtab, elu, interpret=False):
    M = idx.shape[0]
    Mp = -(-M // (NW * C)) * (NW * C)
    idx_flat = jnp.pad(idx, ((0, Mp - M), (0, 0))).reshape(-1)
    c_pad = jnp.pad(cself, (0, Mp - cself.shape[0]))
    fn = _sc_stage_call(Mp, vtab.shape[0], elu, interpret)
    return fn(idx_flat, c_pad, ptab, vtab)


@functools.lru_cache(maxsize=None)
def _assembly_call(B, W, Mp, Np, interpret=False):
    Rt = B // NW
    SLT = Rt * W
    mesh = plsc.VectorSubcoreMesh(core_axis_name="c", subcore_axis_name="s", num_cores=NC, num_subcores=NS)

    def body(b_hbm, e_hbm, n_hbm, out_hbm, b_v, idxs, er, nr, outb, sem):
        wid = lax.axis_index("s") * NC + lax.axis_index("c")
        sbase = pl.multiple_of(wid * SLT, 8)
        pltpu.sync_copy(b_hbm.at[pl.ds(sbase, SLT)], b_v)
        i16 = lax.iota(jnp.int32, L)
        i5 = i16 * W
        for h in range(Rt // L):
            hb = h * L * W
            cols = [plsc.load_gather(b_v, [i5 + (hb + j)]) for j in range(W)]
            lastnz = jnp.full((L,), -1, jnp.int32)
            for j in range(W - 1):
                lastnz = jnp.maximum(lastnz,
                                     jnp.where(cols[j] != 0, j, -1))
            right = lastnz + 1
            idxs[...] = cols[0]
            pltpu.async_copy(e_hbm.at[idxs], er, sem).wait()
            for i in range(L):
                s = (h * L + i) * W
                for k in range(F // L):
                    sl = pl.ds(k * L, L)
                    outb[s, sl] = _elu(er[i, sl])
            for j in range(1, W):
                mf = jnp.where(j < right, 1.0, 0.0)
                idxs[...] = cols[j]
                pltpu.async_copy(n_hbm.at[idxs], nr, sem).wait()
                for i in range(L):
                    w = _bcast16(mf, i)
                    s = (h * L + i) * W + j
                    for k in range(F // L):
                        sl = pl.ds(k * L, L)
                        outb[s, sl] = nr[i, sl] * w
        pltpu.sync_copy(outb, out_hbm.at[pl.ds(sbase, SLT)])

    return pl.kernel(
        body,
        out_type=jax.ShapeDtypeStruct((B * W, F), jnp.float32),
        mesh=mesh,
        scratch_types=[pltpu.VMEM((SLT,), jnp.int32),
                       pltpu.VMEM((L,), jnp.int32),
                       pltpu.VMEM((L, F), jnp.float32),
                       pltpu.VMEM((L, F), jnp.float32),
                       pltpu.VMEM((SLT, F), jnp.float32),
                       pltpu.SemaphoreType.DMA],
        compiler_params=pltpu.CompilerParams(needs_layout_passes=False),
        interpret=interpret,
    )


def kernel(batch_inputs, node_embs, edge_embs, edge_list, node_list,
           Wn0, We0, ae0, an0, Wn1, We1, ae1, an1, _interpret=False):
    N = node_embs.shape[0]
    M = edge_embs.shape[0]
    B, W = batch_inputs.shape

    col = lambda a: a.reshape(F, 1)
    hn0, pe0, q0 = _node_dense_call(N, _interpret)(node_embs, Wn0, col(ae0[F:]),
                                       col(an0[:F]))
    (ce0,) = _edge0_call(M, _interpret)(edge_embs, We0, col(ae0[:F]))
    e1 = _sc_stage(edge_list, ce0[:, 0], pe0[:, 0], hn0, False, _interpret)
    r0, ce1 = _e1_post_call(e1.shape[0], _interpret)(e1, col(an0[F:]), We1, col(ae1[:F]))
    x = _sc_stage(node_list, q0[:, 0], r0[:, 0], e1, True, _interpret)
    hn1, pe1, q1 = _node_dense_call(x.shape[0], _interpret)(x, Wn1, col(ae1[F:]),
                                                col(an1[:F]))
    e2 = _sc_stage(edge_list, ce1[:, 0], pe1[:, 0], hn1, False, _interpret)
    (r1,) = _e2_post_call(e2.shape[0], _interpret)(e2, col(an1[F:]))
    nf = _sc_stage(node_list, q1[:, 0], r1[:, 0], e2, True, _interpret)
    out = _assembly_call(B, W, e2.shape[0], nf.shape[0], _interpret)(
        batch_inputs.reshape(-1), e2, nf)
    return out.reshape(B, W, F)


# speedup vs baseline: 6.6731x; 1.4419x over previous
import functools

import jax
import jax.numpy as jnp
from jax import lax
from jax.experimental import pallas as pl
from jax.experimental.pallas import tpu as pltpu
from jax.experimental.pallas import tpu_sc as plsc

F = 128
A = 4
L = 16
NC, NS = 2, 16
NW = NC * NS
C = 16
ALPHA = 0.2
BR = 1024


def _bcast16(v, lane):
    idx = jnp.full((L, 1), lane, dtype=jnp.int32)
    return lax.gather(
        v, idx,
        lax.GatherDimensionNumbers(offset_dims=(), collapsed_slice_dims=(0,),
                                   start_index_map=(0,)),
        (1,), mode=lax.GatherScatterMode.PROMISE_IN_BOUNDS)


def _elu(v):
    return jnp.where(v > 0, v, jnp.exp(jnp.minimum(v, 0.0)) - 1.0)


def _node_dense_body(x_ref, W_ref, a1_ref, a2_ref, h_ref, p_ref, q_ref):
    h = jnp.dot(x_ref[...], W_ref[...], preferred_element_type=jnp.float32)
    h_ref[...] = h
    p_ref[...] = jnp.dot(h, a1_ref[...], preferred_element_type=jnp.float32)
    q_ref[...] = jnp.dot(h, a2_ref[...], preferred_element_type=jnp.float32)


@functools.lru_cache(maxsize=None)
def _node_dense_call(R, interpret=False):
    grid = (R + BR - 1) // BR
    return pl.pallas_call(
        _node_dense_body,
        grid=(grid,),
        in_specs=[pl.BlockSpec((BR, F), lambda i: (i, 0)),
                  pl.BlockSpec((F, F), lambda i: (0, 0)),
                  pl.BlockSpec((F, 1), lambda i: (0, 0)),
                  pl.BlockSpec((F, 1), lambda i: (0, 0))],
        out_specs=[pl.BlockSpec((BR, F), lambda i: (i, 0)),
                   pl.BlockSpec((BR, 1), lambda i: (i, 0)),
                   pl.BlockSpec((BR, 1), lambda i: (i, 0))],
        out_shape=[jax.ShapeDtypeStruct((R, F), jnp.float32),
                   jax.ShapeDtypeStruct((R, 1), jnp.float32),
                   jax.ShapeDtypeStruct((R, 1), jnp.float32)],
        interpret=interpret,
    )


def _edge0_body(x_ref, W_ref, a_ref, c_ref):
    w = jnp.dot(W_ref[...], a_ref[...], preferred_element_type=jnp.float32)
    c_ref[...] = jnp.dot(x_ref[...], w, preferred_element_type=jnp.float32)


@functools.lru_cache(maxsize=None)
def _edge0_call(R, interpret=False):
    grid = (R + BR - 1) // BR
    return pl.pallas_call(
        _edge0_body,
        grid=(grid,),
        in_specs=[pl.BlockSpec((BR, F), lambda i: (i, 0)),
                  pl.BlockSpec((F, F), lambda i: (0, 0)),
                  pl.BlockSpec((F, 1), lambda i: (0, 0))],
        out_specs=[pl.BlockSpec((BR, 1), lambda i: (i, 0))],
        out_shape=[jax.ShapeDtypeStruct((R, 1), jnp.float32)],
        interpret=interpret,
    )


def _e1_post_body(e_ref, anb_ref, We_ref, aea_ref, r_ref, c_ref):
    e = e_ref[...]
    r_ref[...] = jnp.dot(e, anb_ref[...], preferred_element_type=jnp.float32)
    ye = jnp.where(e > 0, e, jnp.exp(jnp.minimum(e, 0.0)) - 1.0)
    w = jnp.dot(We_ref[...], aea_ref[...], preferred_element_type=jnp.float32)
    c_ref[...] = jnp.dot(ye, w, preferred_element_type=jnp.float32)


@functools.lru_cache(maxsize=None)
def _e1_post_call(R, interpret=False):
    grid = (R + BR - 1) // BR
    return pl.pallas_call(
        _e1_post_body,
        grid=(grid,),
        in_specs=[pl.BlockSpec((BR, F), lambda i: (i, 0)),
                  pl.BlockSpec((F, 1), lambda i: (0, 0)),
                  pl.BlockSpec((F, F), lambda i: (0, 0)),
                  pl.BlockSpec((F, 1), lambda i: (0, 0))],
        out_specs=[pl.BlockSpec((BR, 1), lambda i: (i, 0)),
                   pl.BlockSpec((BR, 1), lambda i: (i, 0))],
        out_shape=[jax.ShapeDtypeStruct((R, 1), jnp.float32),
                   jax.ShapeDtypeStruct((R, 1), jnp.float32)],
        interpret=interpret,
    )


def _e2_post_body(e_ref, anb_ref, r_ref):
    r_ref[...] = jnp.dot(e_ref[...], anb_ref[...],
                         preferred_element_type=jnp.float32)


@functools.lru_cache(maxsize=None)
def _e2_post_call(R, interpret=False):
    grid = (R + BR - 1) // BR
    return pl.pallas_call(
        _e2_post_body,
        grid=(grid,),
        in_specs=[pl.BlockSpec((BR, F), lambda i: (i, 0)),
                  pl.BlockSpec((F, 1), lambda i: (0, 0))],
        out_specs=[pl.BlockSpec((BR, 1), lambda i: (i, 0))],
        out_shape=[jax.ShapeDtypeStruct((R, 1), jnp.float32)],
        interpret=interpret,
    )


@functools.lru_cache(maxsize=None)
def _sc_stage_call(Mp, Ntab, elu, use_cidx=False, interpret=False):
    Tw = Mp // NW
    G = Tw // C
    K = 4 if G % 4 == 0 else 2
    NG = G // K
    CA = C * A
    mesh = plsc.VectorSubcoreMesh(core_axis_name="c", subcore_axis_name="s",
                                  num_cores=NC, num_subcores=NS)

    def core(idx_hbm, c_hbm, cidx_hbm, out_hbm,
             c_all, idx_all, pg, rows, outb, semp, semv, cidx_all,
             p_hbm, v_hbm):
        wid = lax.axis_index("s") * NC + lax.axis_index("c")
        base = pl.multiple_of(wid * Tw, 8)
        if use_cidx:
            pltpu.sync_copy(cidx_hbm.at[pl.ds(base, Tw)], cidx_all)
            pltpu.async_copy(c_hbm.at[cidx_all], c_all, semp[0]).wait()
        else:
            pltpu.sync_copy(c_hbm.at[pl.ds(base, Tw)], c_all)
        pltpu.sync_copy(idx_hbm.at[pl.ds(base * A, Tw * A)], idx_all)
        i16 = lax.iota(jnp.int32, L)
        i4 = i16 * A

        def isl(g):
            return idx_all.at[pl.ds(pl.multiple_of(g * CA, 8), CA)]

        def issue(g, b):
            pltpu.async_copy(p_hbm.at[isl(g)], pg[b], semp[b])
            pltpu.async_copy(v_hbm.at[isl(g)], rows[b], semv[b])

        issue(0, 0)

        def group(g2, carry):
            for k in range(K):
                g = g2 * K + k
                b = k & 1
                nb = (k + 1) & 1
                if k < K - 1:
                    issue(g + 1, nb)
                else:
                    @pl.when(g2 < NG - 1)
                    def _():
                        issue((g2 + 1) * K, 0)
                pltpu.make_async_copy(p_hbm.at[isl(g)], pg[b], semp[b]).wait()
                pltpu.make_async_copy(v_hbm.at[isl(g)], rows[b],
                                      semv[b]).wait()
                for h in range(C // L):
                    cvec = c_all[pl.ds(g * C + h * L, L)]
                    ps = [plsc.load_gather(pg[b], [i4 + (h * L * A + m)])
                          for m in range(A)]
                    ss = [cvec + p for p in ps]
                    ss = [jnp.where(s > 0, s, ALPHA * s) for s in ss]
                    mx = jnp.maximum(jnp.maximum(ss[0], ss[1]),
                                     jnp.maximum(ss[2], ss[3]))
                    es = [jnp.exp(s - mx) for s in ss]
                    tot = (es[0] + es[1]) + (es[2] + es[3])
                    att = [e / tot for e in es]
                    for i in range(L):
                        ii = h * L + i
                        ws = [_bcast16(att[m], i) for m in range(A)]
                        for kk in range(F // L):
                            sl = pl.ds(kk * L, L)
                            acc = ((rows[b][ii * A + 0, sl] * ws[0]
                                    + rows[b][ii * A + 1, sl] * ws[1])
                                   + (rows[b][ii * A + 2, sl] * ws[2]
                                      + rows[b][ii * A + 3, sl] * ws[3]))
                            if elu:
                                acc = _elu(acc)
                            outb[k * C + ii, sl] = acc
            pltpu.sync_copy(outb,
                            out_hbm.at[pl.ds(base + g2 * (K * C), K * C)])
            return carry

        lax.fori_loop(0, NG, group, 0)

    if use_cidx:
        def body(idx_hbm, c_hbm, p_hbm, v_hbm, cidx_hbm, out_hbm,
                 c_all, idx_all, pg0, pg1, rows0, rows1, outb,
                 semp0, semp1, semv0, semv1, cidx_all):
            core(idx_hbm, c_hbm, cidx_hbm, out_hbm, c_all, idx_all,
                 (pg0, pg1), (rows0, rows1), outb, (semp0, semp1),
                 (semv0, semv1), cidx_all, p_hbm, v_hbm)
    else:
        def body(idx_hbm, c_hbm, p_hbm, v_hbm, out_hbm,
                 c_all, idx_all, pg0, pg1, rows0, rows1, outb,
                 semp0, semp1, semv0, semv1):
            core(idx_hbm, c_hbm, None, out_hbm, c_all, idx_all,
                 (pg0, pg1), (rows0, rows1), outb, (semp0, semp1),
                 (semv0, semv1), None, p_hbm, v_hbm)

    return pl.kernel(
        body,
        out_type=jax.ShapeDtypeStruct((Mp, F), jnp.float32),
        mesh=mesh,
        scratch_types=[pltpu.VMEM((Tw,), jnp.float32),
                       pltpu.VMEM((Tw * A,), jnp.int32),
                       pltpu.VMEM((CA,), jnp.float32),
                       pltpu.VMEM((CA,), jnp.float32),
                       pltpu.VMEM((CA, F), jnp.float32),
                       pltpu.VMEM((CA, F), jnp.float32),
                       pltpu.VMEM((K * C, F), jnp.float32),
                       pltpu.SemaphoreType.DMA,
                       pltpu.SemaphoreType.DMA,
                       pltpu.SemaphoreType.DMA,
                       pltpu.SemaphoreType.DMA]
                      + ([pltpu.VMEM((Tw,), jnp.int32)] if use_cidx else []),
        compiler_params=pltpu.CompilerParams(needs_layout_passes=False),
        interpret=interpret,
    )


def _sc_stage(idx, cself, ptab, vtab, elu, interpret=False):
    M = idx.shape[0]
    Mp = -(-M // (NW * C)) * (NW * C)
    idx_flat = jnp.pad(idx, ((0, Mp - M), (0, 0))).reshape(-1)
    c_pad = jnp.pad(cself, (0, Mp - cself.shape[0]))
    fn = _sc_stage_call(Mp, vtab.shape[0], elu, interpret)
    return fn(idx_flat, c_pad, ptab, vtab)


@functools.lru_cache(maxsize=None)
def _prep_call(B, W, interpret=False):
    Rt = B // NW
    SLT = Rt * W
    NE = Rt * (W - 1)
    NR = NE * A
    E1 = B + B * (W - 1) * A
    mesh = plsc.VectorSubcoreMesh(core_axis_name="c", subcore_axis_name="s",
                                  num_cores=NC, num_subcores=NS)

    def body(b_hbm, nlf_hbm, elf_hbm, eidx_hbm, mem1_hbm, bnode_hbm,
             b_v, ebuf, m0buf, mrows0, nbuf, fidx, nodereg, midx1, memreg,
             sem):
        wid = lax.axis_index("s") * NC + lax.axis_index("c")
        pltpu.sync_copy(b_hbm.at[pl.ds(pl.multiple_of(wid * SLT, 8), SLT)],
                        b_v)
        i16 = lax.iota(jnp.int32, L)
        i5 = i16 * W
        i4 = i16 * A
        for h in range(Rt // L):
            c0 = plsc.load_gather(b_v, [i5 + h * L * W])
            ebuf[pl.ds(h * L, L)] = c0
            for m in range(A):
                plsc.store_scatter(m0buf, [i4 + (h * L * A + m)],
                                   c0 * A + m)
        pltpu.sync_copy(ebuf,
                        eidx_hbm.at[pl.ds(pl.multiple_of(wid * Rt, 8), Rt)])
        pltpu.async_copy(elf_hbm.at[m0buf], mrows0, sem).wait()
        pltpu.sync_copy(mrows0, mem1_hbm.at[
            pl.ds(pl.multiple_of(wid * (Rt * A), 8), Rt * A)])
        for h in range(Rt // L):
            for j in range(1, W):
                cj = plsc.load_gather(b_v, [i5 + (h * L * W + j)])
                plsc.store_scatter(nbuf, [i4 + (h * L * A + (j - 1))], cj)
        pltpu.sync_copy(nbuf, bnode_hbm.at[
            pl.ds(pl.multiple_of(wid * NE, 8), NE)])
        for hh in range(NE // L):
            nv = nbuf[pl.ds(hh * L, L)]
            for d in range(A):
                plsc.store_scatter(fidx, [i4 + (hh * L * A + d)],
                                   nv * A + d)
        pltpu.async_copy(nlf_hbm.at[fidx], nodereg, sem).wait()
        pltpu.sync_copy(nodereg, eidx_hbm.at[
            pl.ds(pl.multiple_of(B + wid * NR, 8), NR)])
        for hh in range(NR // L):
            ev = nodereg[pl.ds(hh * L, L)]
            for m in range(A):
                plsc.store_scatter(midx1, [i4 + (hh * L * A + m)],
                                   ev * A + m)
        pltpu.async_copy(elf_hbm.at[midx1], memreg, sem).wait()
        pltpu.sync_copy(memreg, mem1_hbm.at[
            pl.ds(pl.multiple_of(B * A + wid * (NR * A), 8), NR * A)])

    return pl.kernel(
        body,
        out_type=(jax.ShapeDtypeStruct((E1,), jnp.int32),
                  jax.ShapeDtypeStruct((E1 * A,), jnp.int32),
                  jax.ShapeDtypeStruct((B * (W - 1),), jnp.int32)),
        mesh=mesh,
        scratch_types=[pltpu.VMEM((SLT,), jnp.int32),
                       pltpu.VMEM((Rt,), jnp.int32),
                       pltpu.VMEM((Rt * A,), jnp.int32),
                       pltpu.VMEM((Rt * A,), jnp.int32),
                       pltpu.VMEM((NE,), jnp.int32),
                       pltpu.VMEM((NR,), jnp.int32),
                       pltpu.VMEM((NR,), jnp.int32),
                       pltpu.VMEM((NR * A,), jnp.int32),
                       pltpu.VMEM((NR * A,), jnp.int32),
                       pltpu.SemaphoreType.DMA],
        compiler_params=pltpu.CompilerParams(needs_layout_passes=False),
        interpret=interpret,
    )


@functools.lru_cache(maxsize=None)
def _assembly_call(B, W, interpret=False):
    Rt = B // NW
    SLT = Rt * W
    NE = Rt * (W - 1)
    mesh = plsc.VectorSubcoreMesh(core_axis_name="c", subcore_axis_name="s",
                                  num_cores=NC, num_subcores=NS)

    def body(b_hbm, e_hbm, n_hbm, out_hbm, b_v, er, nr, outb, sem):
        wid = lax.axis_index("s") * NC + lax.axis_index("c")
        sbase = pl.multiple_of(wid * SLT, 8)
        pltpu.sync_copy(b_hbm.at[pl.ds(sbase, SLT)], b_v)
        pltpu.sync_copy(e_hbm.at[pl.ds(pl.multiple_of(wid * Rt, 8), Rt)], er)
        pltpu.sync_copy(n_hbm.at[pl.ds(pl.multiple_of(wid * NE, 8), NE)], nr)
        i16 = lax.iota(jnp.int32, L)
        i5 = i16 * W
        for h in range(Rt // L):
            hb = h * L * W
            cols = [plsc.load_gather(b_v, [i5 + (hb + j)])
                    for j in range(W - 1)]
            lastnz = jnp.full((L,), -1, jnp.int32)
            for j in range(W - 1):
                lastnz = jnp.maximum(lastnz,
                                     jnp.where(cols[j] != 0, j, -1))
            right = lastnz + 1
            for i in range(L):
                r = h * L + i
                s = r * W
                for k in range(F // L):
                    sl = pl.ds(k * L, L)
                    outb[s, sl] = _elu(er[r, sl])
            for j in range(1, W):
                mf = jnp.where(j < right, 1.0, 0.0)
                for i in range(L):
                    r = h * L + i
                    w = _bcast16(mf, i)
                    s = r * W + j
                    for k in range(F // L):
                        sl = pl.ds(k * L, L)
                        outb[s, sl] = nr[r * A + (j - 1), sl] * w
        pltpu.sync_copy(outb, out_hbm.at[pl.ds(sbase, SLT)])

    return pl.kernel(
        body,
        out_type=jax.ShapeDtypeStruct((B * W, F), jnp.float32),
        mesh=mesh,
        scratch_types=[pltpu.VMEM((SLT,), jnp.int32),
                       pltpu.VMEM((Rt, F), jnp.float32),
                       pltpu.VMEM((NE, F), jnp.float32),
                       pltpu.VMEM((SLT, F), jnp.float32),
                       pltpu.SemaphoreType.DMA],
        compiler_params=pltpu.CompilerParams(needs_layout_passes=False),
        interpret=interpret,
    )


def kernel(batch_inputs, node_embs, edge_embs, edge_list, node_list,
           Wn0, We0, ae0, an0, Wn1, We1, ae1, an1, _interpret=False):
    N = node_embs.shape[0]
    M = edge_embs.shape[0]
    B, W = batch_inputs.shape
    E1 = B + B * (W - 1) * A
    E4 = B * (W - 1)

    col = lambda a: a.reshape(F, 1)
    batch_flat = batch_inputs.reshape(-1)
    eidx, mem1, bnode = _prep_call(B, W, _interpret)(
        batch_flat, node_list.reshape(-1), edge_list.reshape(-1))
    hn0, pe0, q0 = _node_dense_call(N, _interpret)(node_embs, Wn0,
                                                   col(ae0[F:]),
                                                   col(an0[:F]))
    (ce0,) = _edge0_call(M, _interpret)(edge_embs, We0, col(ae0[:F]))
    e1 = _sc_stage(edge_list, ce0[:, 0], pe0[:, 0], hn0, False, _interpret)
    r0, ce1 = _e1_post_call(e1.shape[0], _interpret)(e1, col(an0[F:]), We1,
                                                     col(ae1[:F]))
    x = _sc_stage(node_list, q0[:, 0], r0[:, 0], e1, True, _interpret)
    hn1, pe1, q1 = _node_dense_call(x.shape[0], _interpret)(x, Wn1,
                                                            col(ae1[F:]),
                                                            col(an1[:F]))
    e2s = _sc_stage_call(E1, hn1.shape[0], False, True, _interpret)(
        mem1, ce1[:, 0], pe1[:, 0], hn1, eidx)
    (r1s,) = _e2_post_call(E1, _interpret)(e2s, col(an1[F:]))
    midx4 = B + jnp.arange(E4 * A, dtype=jnp.int32)
    nf4 = _sc_stage_call(E4, E1, True, True, _interpret)(
        midx4, q1[:, 0], r1s[:, 0], e2s, bnode)
    out = _assembly_call(B, W, _interpret)(batch_flat, e2s, nf4)
    return out.reshape(B, W, F)


# speedup vs baseline: 7.2929x; 1.0929x over previous
import functools

import jax
import jax.numpy as jnp
from jax import lax
from jax.experimental import pallas as pl
from jax.experimental.pallas import tpu as pltpu
from jax.experimental.pallas import tpu_sc as plsc

F = 128
A = 4
L = 16
NC, NS = 2, 16
NW = NC * NS
C = 16
ALPHA = 0.2
BR = 1024


def _bcast16(v, lane):
    idx = jnp.full((L, 1), lane, dtype=jnp.int32)
    return lax.gather(
        v, idx,
        lax.GatherDimensionNumbers(offset_dims=(), collapsed_slice_dims=(0,),
                                   start_index_map=(0,)),
        (1,), mode=lax.GatherScatterMode.PROMISE_IN_BOUNDS)


def _elu(v):
    return jnp.where(v > 0, v, jnp.exp(jnp.minimum(v, 0.0)) - 1.0)


def _node_dense_body(x_ref, W_ref, a1_ref, a2_ref, h_ref, p_ref, q_ref):
    h = jnp.dot(x_ref[...], W_ref[...], preferred_element_type=jnp.float32)
    h_ref[...] = h
    p_ref[...] = jnp.dot(h, a1_ref[...], preferred_element_type=jnp.float32)
    q_ref[...] = jnp.dot(h, a2_ref[...], preferred_element_type=jnp.float32)


def _dense0_body(x_ref, e_ref, W_ref, a1_ref, a2_ref, We_ref, aea_ref,
                 h_ref, p_ref, q_ref, c_ref):
    h = jnp.dot(x_ref[...], W_ref[...], preferred_element_type=jnp.float32)
    h_ref[...] = h
    p_ref[...] = jnp.dot(h, a1_ref[...], preferred_element_type=jnp.float32)
    q_ref[...] = jnp.dot(h, a2_ref[...], preferred_element_type=jnp.float32)
    w = jnp.dot(We_ref[...], aea_ref[...], preferred_element_type=jnp.float32)
    c_ref[...] = jnp.dot(e_ref[...], w, preferred_element_type=jnp.float32)


@functools.lru_cache(maxsize=None)
def _dense0_call(R, interpret=False):
    grid = (R + BR - 1) // BR
    return pl.pallas_call(
        _dense0_body,
        grid=(grid,),
        in_specs=[pl.BlockSpec((BR, F), lambda i: (i, 0)),
                  pl.BlockSpec((BR, F), lambda i: (i, 0)),
                  pl.BlockSpec((F, F), lambda i: (0, 0)),
                  pl.BlockSpec((F, 1), lambda i: (0, 0)),
                  pl.BlockSpec((F, 1), lambda i: (0, 0)),
                  pl.BlockSpec((F, F), lambda i: (0, 0)),
                  pl.BlockSpec((F, 1), lambda i: (0, 0))],
        out_specs=[pl.BlockSpec((BR, F), lambda i: (i, 0)),
                   pl.BlockSpec((BR, 1), lambda i: (i, 0)),
                   pl.BlockSpec((BR, 1), lambda i: (i, 0)),
                   pl.BlockSpec((BR, 1), lambda i: (i, 0))],
        out_shape=[jax.ShapeDtypeStruct((R, F), jnp.float32),
                   jax.ShapeDtypeStruct((R, 1), jnp.float32),
                   jax.ShapeDtypeStruct((R, 1), jnp.float32),
                   jax.ShapeDtypeStruct((R, 1), jnp.float32)],
        interpret=interpret,
    )


@functools.lru_cache(maxsize=None)
def _node_dense_call(R, interpret=False):
    grid = (R + BR - 1) // BR
    return pl.pallas_call(
        _node_dense_body,
        grid=(grid,),
        in_specs=[pl.BlockSpec((BR, F), lambda i: (i, 0)),
                  pl.BlockSpec((F, F), lambda i: (0, 0)),
                  pl.BlockSpec((F, 1), lambda i: (0, 0)),
                  pl.BlockSpec((F, 1), lambda i: (0, 0))],
        out_specs=[pl.BlockSpec((BR, F), lambda i: (i, 0)),
                   pl.BlockSpec((BR, 1), lambda i: (i, 0)),
                   pl.BlockSpec((BR, 1), lambda i: (i, 0))],
        out_shape=[jax.ShapeDtypeStruct((R, F), jnp.float32),
                   jax.ShapeDtypeStruct((R, 1), jnp.float32),
                   jax.ShapeDtypeStruct((R, 1), jnp.float32)],
        interpret=interpret,
    )


def _edge0_body(x_ref, W_ref, a_ref, c_ref):
    w = jnp.dot(W_ref[...], a_ref[...], preferred_element_type=jnp.float32)
    c_ref[...] = jnp.dot(x_ref[...], w, preferred_element_type=jnp.float32)


@functools.lru_cache(maxsize=None)
def _edge0_call(R, interpret=False):
    grid = (R + BR - 1) // BR
    return pl.pallas_call(
        _edge0_body,
        grid=(grid,),
        in_specs=[pl.BlockSpec((BR, F), lambda i: (i, 0)),
                  pl.BlockSpec((F, F), lambda i: (0, 0)),
                  pl.BlockSpec((F, 1), lambda i: (0, 0))],
        out_specs=[pl.BlockSpec((BR, 1), lambda i: (i, 0))],
        out_shape=[jax.ShapeDtypeStruct((R, 1), jnp.float32)],
        interpret=interpret,
    )


def _e1_post_body(e_ref, anb_ref, We_ref, aea_ref, r_ref, c_ref):
    e = e_ref[...]
    r_ref[...] = jnp.dot(e, anb_ref[...], preferred_element_type=jnp.float32)
    ye = jnp.where(e > 0, e, jnp.exp(jnp.minimum(e, 0.0)) - 1.0)
    w = jnp.dot(We_ref[...], aea_ref[...], preferred_element_type=jnp.float32)
    c_ref[...] = jnp.dot(ye, w, preferred_element_type=jnp.float32)


@functools.lru_cache(maxsize=None)
def _e1_post_call(R, interpret=False):
    grid = (R + BR - 1) // BR
    return pl.pallas_call(
        _e1_post_body,
        grid=(grid,),
        in_specs=[pl.BlockSpec((BR, F), lambda i: (i, 0)),
                  pl.BlockSpec((F, 1), lambda i: (0, 0)),
                  pl.BlockSpec((F, F), lambda i: (0, 0)),
                  pl.BlockSpec((F, 1), lambda i: (0, 0))],
        out_specs=[pl.BlockSpec((BR, 1), lambda i: (i, 0)),
                   pl.BlockSpec((BR, 1), lambda i: (i, 0))],
        out_shape=[jax.ShapeDtypeStruct((R, 1), jnp.float32),
                   jax.ShapeDtypeStruct((R, 1), jnp.float32)],
        interpret=interpret,
    )


def _e2_post_body(e_ref, anb_ref, r_ref):
    r_ref[...] = jnp.dot(e_ref[...], anb_ref[...],
                         preferred_element_type=jnp.float32)


@functools.lru_cache(maxsize=None)
def _e2_post_call(R, interpret=False):
    grid = (R + BR - 1) // BR
    return pl.pallas_call(
        _e2_post_body,
        grid=(grid,),
        in_specs=[pl.BlockSpec((BR, F), lambda i: (i, 0)),
                  pl.BlockSpec((F, 1), lambda i: (0, 0))],
        out_specs=[pl.BlockSpec((BR, 1), lambda i: (i, 0))],
        out_shape=[jax.ShapeDtypeStruct((R, 1), jnp.float32)],
        interpret=interpret,
    )


@functools.lru_cache(maxsize=None)
def _sc_stage_call(Mp, Ntab, elu, use_cidx=False, C=C, interpret=False):
    Tw = Mp // NW
    G = Tw // C
    K = 4 if G % 4 == 0 else 2
    NG = G // K
    CA = C * A
    mesh = plsc.VectorSubcoreMesh(core_axis_name="c", subcore_axis_name="s",
                                  num_cores=NC, num_subcores=NS)

    def core(idx_hbm, c_hbm, cidx_hbm, out_hbm,
             c_all, idx_all, pg, rows, outb, semp, semv, cidx_all,
             p_hbm, v_hbm):
        wid = lax.axis_index("s") * NC + lax.axis_index("c")
        base = pl.multiple_of(wid * Tw, 8)
        if use_cidx:
            pltpu.sync_copy(cidx_hbm.at[pl.ds(base, Tw)], cidx_all)
            pltpu.async_copy(c_hbm.at[cidx_all], c_all, semp[0]).wait()
        else:
            pltpu.sync_copy(c_hbm.at[pl.ds(base, Tw)], c_all)
        pltpu.sync_copy(idx_hbm.at[pl.ds(base * A, Tw * A)], idx_all)
        i16 = lax.iota(jnp.int32, L)
        i4 = i16 * A

        def isl(g):
            return idx_all.at[pl.ds(pl.multiple_of(g * CA, 8), CA)]

        def issue(g, b):
            pltpu.async_copy(p_hbm.at[isl(g)], pg[b], semp[b])
            pltpu.async_copy(v_hbm.at[isl(g)], rows[b], semv[b])

        issue(0, 0)

        def group(g2, carry):
            for k in range(K):
                g = g2 * K + k
                b = k & 1
                nb = (k + 1) & 1
                if k < K - 1:
                    issue(g + 1, nb)
                else:
                    @pl.when(g2 < NG - 1)
                    def _():
                        issue((g2 + 1) * K, 0)
                pltpu.make_async_copy(p_hbm.at[isl(g)], pg[b], semp[b]).wait()
                pltpu.make_async_copy(v_hbm.at[isl(g)], rows[b],
                                      semv[b]).wait()
                for h in range(C // L):
                    cvec = c_all[pl.ds(g * C + h * L, L)]
                    ps = [plsc.load_gather(pg[b], [i4 + (h * L * A + m)])
                          for m in range(A)]
                    ss = [cvec + p for p in ps]
                    ss = [jnp.where(s > 0, s, ALPHA * s) for s in ss]
                    mx = jnp.maximum(jnp.maximum(ss[0], ss[1]),
                                     jnp.maximum(ss[2], ss[3]))
                    es = [jnp.exp(s - mx) for s in ss]
                    tot = (es[0] + es[1]) + (es[2] + es[3])
                    att = [e / tot for e in es]
                    for i in range(L):
                        ii = h * L + i
                        ws = [_bcast16(att[m], i) for m in range(A)]
                        for kk in range(F // L):
                            sl = pl.ds(kk * L, L)
                            acc = ((rows[b][ii * A + 0, sl] * ws[0]
                                    + rows[b][ii * A + 1, sl] * ws[1])
                                   + (rows[b][ii * A + 2, sl] * ws[2]
                                      + rows[b][ii * A + 3, sl] * ws[3]))
                            if elu:
                                acc = _elu(acc)
                            outb[k * C + ii, sl] = acc
            pltpu.sync_copy(outb,
                            out_hbm.at[pl.ds(base + g2 * (K * C), K * C)])
            return carry

        lax.fori_loop(0, NG, group, 0)

    if use_cidx:
        def body(idx_hbm, c_hbm, p_hbm, v_hbm, cidx_hbm, out_hbm,
                 c_all, idx_all, pg0, pg1, rows0, rows1, outb,
                 semp0, semp1, semv0, semv1, cidx_all):
            core(idx_hbm, c_hbm, cidx_hbm, out_hbm, c_all, idx_all,
                 (pg0, pg1), (rows0, rows1), outb, (semp0, semp1),
                 (semv0, semv1), cidx_all, p_hbm, v_hbm)
    else:
        def body(idx_hbm, c_hbm, p_hbm, v_hbm, out_hbm,
                 c_all, idx_all, pg0, pg1, rows0, rows1, outb,
                 semp0, semp1, semv0, semv1):
            core(idx_hbm, c_hbm, None, out_hbm, c_all, idx_all,
                 (pg0, pg1), (rows0, rows1), outb, (semp0, semp1),
                 (semv0, semv1), None, p_hbm, v_hbm)

    return pl.kernel(
        body,
        out_type=jax.ShapeDtypeStruct((Mp, F), jnp.float32),
        mesh=mesh,
        scratch_types=[pltpu.VMEM((Tw,), jnp.float32),
                       pltpu.VMEM((Tw * A,), jnp.int32),
                       pltpu.VMEM((CA,), jnp.float32),
                       pltpu.VMEM((CA,), jnp.float32),
                       pltpu.VMEM((CA, F), jnp.float32),
                       pltpu.VMEM((CA, F), jnp.float32),
                       pltpu.VMEM((K * C, F), jnp.float32),
                       pltpu.SemaphoreType.DMA,
                       pltpu.SemaphoreType.DMA,
                       pltpu.SemaphoreType.DMA,
                       pltpu.SemaphoreType.DMA]
                      + ([pltpu.VMEM((Tw,), jnp.int32)] if use_cidx else []),
        compiler_params=pltpu.CompilerParams(needs_layout_passes=False),
        interpret=interpret,
    )


def _sc_stage(idx, cself, ptab, vtab, elu, interpret=False):
    CB = 32
    M = idx.shape[0]
    Mp = -(-M // (NW * CB)) * (NW * CB)
    idx_flat = jnp.pad(idx, ((0, Mp - M), (0, 0))).reshape(-1)
    c_pad = jnp.pad(cself, (0, Mp - cself.shape[0]))
    fn = _sc_stage_call(Mp, vtab.shape[0], elu, False, CB, interpret)
    return fn(idx_flat, c_pad, ptab, vtab)


@functools.lru_cache(maxsize=None)
def _prep_call(B, W, interpret=False):
    Rt = B // NW
    SLT = Rt * W
    NE = Rt * (W - 1)
    NR = NE * A
    E1 = B + B * (W - 1) * A
    mesh = plsc.VectorSubcoreMesh(core_axis_name="c", subcore_axis_name="s",
                                  num_cores=NC, num_subcores=NS)

    def body(b_hbm, nlf_hbm, elf_hbm, eidx_hbm, mem1_hbm, bnode_hbm,
             b_v, ebuf, m0buf, mrows0, nbuf, fidx, nodereg, midx1, memreg,
             sem):
        wid = lax.axis_index("s") * NC + lax.axis_index("c")
        pltpu.sync_copy(b_hbm.at[pl.ds(pl.multiple_of(wid * SLT, 8), SLT)],
                        b_v)
        i16 = lax.iota(jnp.int32, L)
        i5 = i16 * W
        i4 = i16 * A
        for h in range(Rt // L):
            c0 = plsc.load_gather(b_v, [i5 + h * L * W])
            ebuf[pl.ds(h * L, L)] = c0
            for m in range(A):
                plsc.store_scatter(m0buf, [i4 + (h * L * A + m)],
                                   c0 * A + m)
        pltpu.sync_copy(ebuf,
                        eidx_hbm.at[pl.ds(pl.multiple_of(wid * Rt, 8), Rt)])
        pltpu.async_copy(elf_hbm.at[m0buf], mrows0, sem).wait()
        pltpu.sync_copy(mrows0, mem1_hbm.at[
            pl.ds(pl.multiple_of(wid * (Rt * A), 8), Rt * A)])
        for h in range(Rt // L):
            for j in range(1, W):
                cj = plsc.load_gather(b_v, [i5 + (h * L * W + j)])
                plsc.store_scatter(nbuf, [i4 + (h * L * A + (j - 1))], cj)
        pltpu.sync_copy(nbuf, bnode_hbm.at[
            pl.ds(pl.multiple_of(wid * NE, 8), NE)])
        for hh in range(NE // L):
            nv = nbuf[pl.ds(hh * L, L)]
            for d in range(A):
                plsc.store_scatter(fidx, [i4 + (hh * L * A + d)],
                                   nv * A + d)
        pltpu.async_copy(nlf_hbm.at[fidx], nodereg, sem).wait()
        pltpu.sync_copy(nodereg, eidx_hbm.at[
            pl.ds(pl.multiple_of(B + wid * NR, 8), NR)])
        for hh in range(NR // L):
            ev = nodereg[pl.ds(hh * L, L)]
            for m in range(A):
                plsc.store_scatter(midx1, [i4 + (hh * L * A + m)],
                                   ev * A + m)
        pltpu.async_copy(elf_hbm.at[midx1], memreg, sem).wait()
        pltpu.sync_copy(memreg, mem1_hbm.at[
            pl.ds(pl.multiple_of(B * A + wid * (NR * A), 8), NR * A)])

    return pl.kernel(
        body,
        out_type=(jax.ShapeDtypeStruct((E1,), jnp.int32),
                  jax.ShapeDtypeStruct((E1 * A,), jnp.int32),
                  jax.ShapeDtypeStruct((B * (W - 1),), jnp.int32)),
        mesh=mesh,
        scratch_types=[pltpu.VMEM((SLT,), jnp.int32),
                       pltpu.VMEM((Rt,), jnp.int32),
                       pltpu.VMEM((Rt * A,), jnp.int32),
                       pltpu.VMEM((Rt * A,), jnp.int32),
                       pltpu.VMEM((NE,), jnp.int32),
                       pltpu.VMEM((NR,), jnp.int32),
                       pltpu.VMEM((NR,), jnp.int32),
                       pltpu.VMEM((NR * A,), jnp.int32),
                       pltpu.VMEM((NR * A,), jnp.int32),
                       pltpu.SemaphoreType.DMA],
        compiler_params=pltpu.CompilerParams(needs_layout_passes=False),
        interpret=interpret,
    )


@functools.lru_cache(maxsize=None)
def _assembly_call(B, W, interpret=False):
    Rt = B // NW
    SLT = Rt * W
    NE = Rt * (W - 1)
    mesh = plsc.VectorSubcoreMesh(core_axis_name="c", subcore_axis_name="s",
                                  num_cores=NC, num_subcores=NS)

    def body(b_hbm, e_hbm, n_hbm, out_hbm, b_v, er, nr, outb, sem):
        wid = lax.axis_index("s") * NC + lax.axis_index("c")
        sbase = pl.multiple_of(wid * SLT, 8)
        pltpu.sync_copy(b_hbm.at[pl.ds(sbase, SLT)], b_v)
        pltpu.sync_copy(e_hbm.at[pl.ds(pl.multiple_of(wid * Rt, 8), Rt)], er)
        pltpu.sync_copy(n_hbm.at[pl.ds(pl.multiple_of(wid * NE, 8), NE)], nr)
        i16 = lax.iota(jnp.int32, L)
        i5 = i16 * W
        for h in range(Rt // L):
            hb = h * L * W
            cols = [plsc.load_gather(b_v, [i5 + (hb + j)])
                    for j in range(W - 1)]
            lastnz = jnp.full((L,), -1, jnp.int32)
            for j in range(W - 1):
                lastnz = jnp.maximum(lastnz,
                                     jnp.where(cols[j] != 0, j, -1))
            right = lastnz + 1
            for i in range(L):
                r = h * L + i
                s = r * W
                for k in range(F // L):
                    sl = pl.ds(k * L, L)
                    outb[s, sl] = _elu(er[r, sl])
            for j in range(1, W):
                mf = jnp.where(j < right, 1.0, 0.0)
                for i in range(L):
                    r = h * L + i
                    w = _bcast16(mf, i)
                    s = r * W + j
                    for k in range(F // L):
                        sl = pl.ds(k * L, L)
                        outb[s, sl] = nr[r * A + (j - 1), sl] * w
        pltpu.sync_copy(outb, out_hbm.at[pl.ds(sbase, SLT)])

    return pl.kernel(
        body,
        out_type=jax.ShapeDtypeStruct((B * W, F), jnp.float32),
        mesh=mesh,
        scratch_types=[pltpu.VMEM((SLT,), jnp.int32),
                       pltpu.VMEM((Rt, F), jnp.float32),
                       pltpu.VMEM((NE, F), jnp.float32),
                       pltpu.VMEM((SLT, F), jnp.float32),
                       pltpu.SemaphoreType.DMA],
        compiler_params=pltpu.CompilerParams(needs_layout_passes=False),
        interpret=interpret,
    )


def kernel(batch_inputs, node_embs, edge_embs, edge_list, node_list,
           Wn0, We0, ae0, an0, Wn1, We1, ae1, an1, _interpret=False):
    N = node_embs.shape[0]
    M = edge_embs.shape[0]
    B, W = batch_inputs.shape
    E1 = B + B * (W - 1) * A
    E4 = B * (W - 1)

    col = lambda a: a.reshape(F, 1)
    batch_flat = batch_inputs.reshape(-1)
    eidx, mem1, bnode = _prep_call(B, W, _interpret)(
        batch_flat, node_list.reshape(-1), edge_list.reshape(-1))
    hn0, pe0, q0, ce0 = _dense0_call(N, _interpret)(
        node_embs, edge_embs, Wn0, col(ae0[F:]), col(an0[:F]), We0,
        col(ae0[:F]))
    e1 = _sc_stage(edge_list, ce0[:, 0], pe0[:, 0], hn0, False, _interpret)
    r0, ce1 = _e1_post_call(e1.shape[0], _interpret)(e1, col(an0[F:]), We1,
                                                     col(ae1[:F]))
    x = _sc_stage(node_list, q0[:, 0], r0[:, 0], e1, True, _interpret)
    hn1, pe1, q1 = _node_dense_call(x.shape[0], _interpret)(x, Wn1,
                                                            col(ae1[F:]),
                                                            col(an1[:F]))
    e2s = _sc_stage_call(E1, hn1.shape[0], False, True, C, _interpret)(
        mem1, ce1[:, 0], pe1[:, 0], hn1, eidx)
    (r1s,) = _e2_post_call(E1, _interpret)(e2s, col(an1[F:]))
    midx4 = B + jnp.arange(E4 * A, dtype=jnp.int32)
    nf4 = _sc_stage_call(E4, E1, True, True, C, _interpret)(
        midx4, q1[:, 0], r1s[:, 0], e2s, bnode)
    out = _assembly_call(B, W, _interpret)(batch_flat, e2s, nf4)
    return out.reshape(B, W, F)
